# Initial kernel scaffold; baseline (speedup 1.0000x reference)
import jax
import jax.numpy as jnp
from jax.experimental import pallas as pl


def kernel(x_enc, x_mark_enc, x_dec, x_mark_dec, w_gate, expert_W, expert_b, head_W, head_b, revin_w, revin_b):
    raise NotImplementedError("write your pallas kernel here")


# speedup vs baseline: 1.4651x; 1.4651x over previous
import functools

import jax
import jax.numpy as jnp
from jax.experimental import pallas as pl
from jax.experimental.pallas import tpu as pltpu

BATCH = 32
SEQ_LEN = 512
PRED_LEN = 336
ENC_IN = 16
D_MODEL = 1024
NUM_EXPERTS = 8
BN = BATCH * ENC_IN


def _fused_body(xt_ref, wg_ref, ew_ref, eb_ref, hw_ref, hb_ref, rv_ref,
                out_ref, y_acc):
    e = pl.program_id(0)
    x = xt_ref[...]
    m = jnp.mean(x, axis=1, keepdims=True)
    xc = x - m
    var = jnp.mean(xc * xc, axis=1, keepdims=True)
    std = jnp.sqrt(var + 1e-5)
    ci = xc / std

    logits = jnp.dot(ci, wg_ref[...], preferred_element_type=jnp.float32)
    io = jax.lax.broadcasted_iota(jnp.int32, (BN, NUM_EXPERTS), 1)
    v1 = jnp.max(logits, axis=1, keepdims=True)
    e1 = jnp.min(jnp.where(logits == v1, io, NUM_EXPERTS), axis=1,
                 keepdims=True)
    l2 = jnp.where(io == e1, -1e30, logits)
    v2 = jnp.max(l2, axis=1, keepdims=True)
    e2 = jnp.min(jnp.where(l2 == v2, io, NUM_EXPERTS), axis=1, keepdims=True)
    g1 = 1.0 / (1.0 + jnp.exp(v2 - v1))
    g2 = 1.0 - g1
    gate_e = g1 * (e1 == e) + g2 * (e2 == e)

    eo = jnp.maximum(
        jnp.dot(ci, ew_ref[0], preferred_element_type=jnp.float32)
        + eb_ref[0], 0.0)

    @pl.when(e == 0)
    def _():
        y_acc[...] = gate_e * eo

    @pl.when(e > 0)
    def _():
        y_acc[...] += gate_e * eo

    @pl.when(e == NUM_EXPERTS - 1)
    def _():
        z = jnp.dot(y_acc[...], hw_ref[...],
                    preferred_element_type=jnp.float32) + hb_ref[...]
        rw = rv_ref[:, 0:1]
        rb = rv_ref[:, 1:2]
        out_ref[...] = (z * rw + rb) * std + m


@jax.jit
def kernel(x_enc, x_mark_enc, x_dec, x_mark_dec, w_gate, expert_W, expert_b,
           head_W, head_b, revin_w, revin_b):
    xt = jnp.transpose(x_enc, (0, 2, 1)).reshape(BN, SEQ_LEN)
    rv = jnp.stack([jnp.tile(revin_w, BATCH), jnp.tile(revin_b, BATCH)],
                   axis=1)

    out_tok = pl.pallas_call(
        _fused_body,
        grid=(NUM_EXPERTS,),
        in_specs=[
            pl.BlockSpec((BN, SEQ_LEN), lambda e: (0, 0)),
            pl.BlockSpec((SEQ_LEN, NUM_EXPERTS), lambda e: (0, 0)),
            pl.BlockSpec((1, SEQ_LEN, D_MODEL), lambda e: (e, 0, 0)),
            pl.BlockSpec((1, 1, D_MODEL), lambda e: (e, 0, 0)),
            pl.BlockSpec((D_MODEL, PRED_LEN), lambda e: (0, 0)),
            pl.BlockSpec((1, PRED_LEN), lambda e: (0, 0)),
            pl.BlockSpec((BN, 2), lambda e: (0, 0)),
        ],
        out_specs=pl.BlockSpec((BN, PRED_LEN), lambda e: (0, 0)),
        out_shape=jax.ShapeDtypeStruct((BN, PRED_LEN), jnp.float32),
        scratch_shapes=[pltpu.VMEM((BN, D_MODEL), jnp.float32)],
        compiler_params=pltpu.CompilerParams(
            dimension_semantics=("arbitrary",)),
    )(xt, w_gate, expert_W, expert_b.reshape(NUM_EXPERTS, 1, D_MODEL),
      head_W, head_b.reshape(1, PRED_LEN), rv)

    return out_tok.reshape(BATCH, ENC_IN, PRED_LEN).transpose(0, 2, 1)
